# baseline (speedup 1.0000x reference)
import functools

import jax
import jax.numpy as jnp
import numpy as np
from jax.experimental import pallas as pl
from jax.experimental.pallas import tpu as pltpu

F = 671
H1 = 128
OUT = 30
C1 = 6
K1 = 5
K2 = 3
NEG = 0.01
NB = 6
XW = 1024
CS = 896
TILE_B = 512


def _leaky(v):
    return jnp.maximum(v, NEG * v)


def _conv_mlp_kernel(x_ref, m1_ref, m2_ref, b1_ref, b2_ref,
                     w3_ref, b3_ref, w4_ref, b4_ref,
                     o_ref, xext, a1s, y2s):
    tb = xext.shape[0]

    xext[:, 0:128] = jnp.zeros((tb, 128), jnp.bfloat16)
    xext[:, 128 + F:XW] = jnp.zeros((tb, XW - 128 - F), jnp.bfloat16)
    xext[:, 128:128 + F] = x_ref[...].astype(jnp.bfloat16)

    for c in range(C1):
        a1s[:, c * CS + 768:(c + 1) * CS] = jnp.zeros((tb, 128), jnp.bfloat16)

    for j in range(NB):
        r = jnp.dot(xext[:, 128 * j:128 * j + 256], m1_ref[...],
                    preferred_element_type=jnp.float32)
        r = _leaky(r + b1_ref[...])
        f = jax.lax.broadcasted_iota(jnp.int32, (tb, 128), 1) + (128 * j - 64)
        valid = (f >= 0) & (f < F)
        for c in range(C1):
            a1s[:, c * CS + 128 * j:c * CS + 128 * (j + 1)] = jnp.where(
                valid, r[:, 128 * c:128 * (c + 1)], 0.0).astype(jnp.bfloat16)

    for j in range(NB):
        acc = jnp.dot(a1s[:, 128 * j:128 * j + 256], m2_ref[0:256, :],
                      preferred_element_type=jnp.float32)
        for c in range(1, C1):
            acc = acc + jnp.dot(
                a1s[:, c * CS + 128 * j:c * CS + 128 * j + 256],
                m2_ref[c * 256:(c + 1) * 256, :],
                preferred_element_type=jnp.float32)
        y2s[:, 128 * j:128 * (j + 1)] = _leaky(acc + b2_ref[...])

    y3 = _leaky(jnp.dot(y2s[...], w3_ref[...],
                        preferred_element_type=jnp.float32) + b3_ref[...])
    o_ref[...] = jnp.dot(y3, w4_ref[...],
                         preferred_element_type=jnp.float32) + b4_ref[...]


def _np_masks():
    t = np.arange(256)[:, None]
    l1 = np.arange(128)[None, :]
    band1 = [(t - l1 == 62 + d).astype(np.float32) for d in range(K1)]
    band2 = [(t - l1 == 63 + k).astype(np.float32) for k in range(K2)]
    return band1, band2


_BAND1, _BAND2 = _np_masks()


def _banded_weights(w1c, w2c):
    m1 = jnp.concatenate(
        [sum(w1c[oc, d] * _BAND1[d] for d in range(K1)) for oc in range(C1)],
        axis=1)
    m2 = jnp.concatenate(
        [sum(w2c[oc, k] * _BAND2[k] for k in range(K2)) for oc in range(C1)],
        axis=0)
    return m1.astype(jnp.bfloat16), m2.astype(jnp.bfloat16)


def kernel(x, w1c, b1, w2c, b2, w3p, b3r, w4t, b4r):
    x = x.astype(jnp.float32)
    B = x.shape[0]
    tile_b = min(TILE_B, B)
    n_tiles = pl.cdiv(B, tile_b)

    m1, m2 = _banded_weights(w1c, w2c)
    b1r = jnp.repeat(b1, 128).reshape(1, C1 * 128)
    b2r = jnp.broadcast_to(b2.reshape(1, 1), (1, 128))

    res = lambda shape: pl.BlockSpec(shape, lambda i: (0, 0))
    return pl.pallas_call(
        _conv_mlp_kernel,
        out_shape=jax.ShapeDtypeStruct((B, OUT), jnp.float32),
        grid=(n_tiles,),
        in_specs=[pl.BlockSpec((tile_b, F), lambda i: (i, 0)),
                  res((256, C1 * 128)),
                  res((C1 * 256, 128)),
                  res((1, C1 * 128)),
                  res((1, 128)),
                  res((768, H1)),
                  res((1, H1)),
                  res((H1, OUT)),
                  res((1, OUT))],
        out_specs=pl.BlockSpec((tile_b, OUT), lambda i: (i, 0)),
        scratch_shapes=[pltpu.VMEM((tile_b, XW), jnp.bfloat16),
                        pltpu.VMEM((tile_b, C1 * CS), jnp.bfloat16),
                        pltpu.VMEM((tile_b, NB * 128), jnp.float32)],
        compiler_params=pltpu.CompilerParams(
            dimension_semantics=("parallel",)),
    )(x, m1, m2, b1r, b2r, w3p, b3r, w4t, b4r)


# speedup vs baseline: 1.2966x; 1.2966x over previous
import jax
import jax.numpy as jnp
from jax.experimental import pallas as pl
from jax.experimental.pallas import tpu as pltpu

F = 671
H1 = 128
OUT = 30
C1 = 6
NV = 2
NM = C1 - NV
K1 = 5
K2 = 3
NEG = 0.01
NB = 6
XW = 1024
CS = 896
TILE_B = 512


def _leaky(v):
    return jnp.maximum(v, NEG * v)


def _weights_kernel(w1_ref, w2_ref, m1_ref, m2_ref):
    t = jax.lax.broadcasted_iota(jnp.int32, (256, 128), 0)
    l = jax.lax.broadcasted_iota(jnp.int32, (256, 128), 1)
    d = t - l
    for oc in range(C1):
        m = jnp.where(d == 62, w1_ref[oc, 0], 0.0)
        for k in range(1, K1):
            m = m + jnp.where(d == 62 + k, w1_ref[oc, k], 0.0)
        m1_ref[:, 128 * oc:128 * (oc + 1)] = m
    for i in range(NM):
        m = jnp.where(d == 63, w2_ref[NV + i, 0], 0.0)
        for k in range(1, K2):
            m = m + jnp.where(d == 63 + k, w2_ref[NV + i, k], 0.0)
        m2_ref[256 * i:256 * (i + 1), :] = m


def _conv_mlp_kernel(w1_ref, b1_ref, w2_ref, b2_ref,
                     x_ref, m1_ref, m2_ref,
                     w3_ref, b3_ref, w4_ref, b4_ref,
                     o_ref, xext, a1s, sks, y2s):
    tb = xext.shape[0]

    xext[:, 0:128] = jnp.zeros((tb, 128), jnp.float32)
    xext[:, 128 + F:XW] = jnp.zeros((tb, XW - 128 - F), jnp.float32)
    xext[:, 128:128 + F] = x_ref[...]

    for c in range(NM):
        a1s[:, c * CS + 768:(c + 1) * CS] = jnp.zeros((tb, 128), jnp.float32)
    for k in range(K2):
        sks[:, k * CS + 768:(k + 1) * CS] = jnp.zeros((tb, 128), jnp.float32)

    for j in range(NB):
        r = jnp.dot(xext[:, 128 * j:128 * j + 256], m1_ref[...],
                    preferred_element_type=jnp.float32)
        if j == 0 or j == NB - 1:
            f = jax.lax.broadcasted_iota(jnp.int32, (tb, 128), 1) + (128 * j - 64)
            valid = (f >= 0) & (f < F)
        else:
            valid = None

        def act(c):
            blk = _leaky(r[:, 128 * c:128 * (c + 1)] + b1_ref[c])
            if valid is not None:
                blk = jnp.where(valid, blk, 0.0)
            return blk

        a0 = act(0)
        a1 = act(1)
        for k in range(K2):
            sks[:, k * CS + 128 * j:k * CS + 128 * (j + 1)] = (
                w2_ref[0, k] * a0 + w2_ref[1, k] * a1)
        for c in range(NM):
            a1s[:, c * CS + 128 * j:c * CS + 128 * (j + 1)] = act(NV + c)

    for j in range(NB):
        acc = jnp.dot(a1s[:, 128 * j:128 * j + 256], m2_ref[0:256, :],
                      preferred_element_type=jnp.float32)
        for c in range(1, NM):
            acc = acc + jnp.dot(
                a1s[:, c * CS + 128 * j:c * CS + 128 * j + 256],
                m2_ref[c * 256:(c + 1) * 256, :],
                preferred_element_type=jnp.float32)
        for k in range(K2):
            acc = acc + sks[:, k * CS + 128 * j + 63 + k:
                            k * CS + 128 * j + 63 + k + 128]
        y2s[:, 128 * j:128 * (j + 1)] = _leaky(acc + b2_ref[0])

    y3 = _leaky(jnp.dot(y2s[...], w3_ref[...],
                        preferred_element_type=jnp.float32) + b3_ref[...])
    o_ref[...] = jnp.dot(y3, w4_ref[...],
                         preferred_element_type=jnp.float32) + b4_ref[...]


def kernel(x, w1c, b1, w2c, b2, w3p, b3r, w4t, b4r):
    x = x.astype(jnp.float32)
    B = x.shape[0]
    tile_b = min(TILE_B, B)
    n_tiles = pl.cdiv(B, tile_b)

    smem = pl.BlockSpec(memory_space=pltpu.MemorySpace.SMEM)
    m1, m2 = pl.pallas_call(
        _weights_kernel,
        out_shape=(jax.ShapeDtypeStruct((256, C1 * 128), jnp.float32),
                   jax.ShapeDtypeStruct((NM * 256, 128), jnp.float32)),
        in_specs=[smem, smem],
    )(w1c, w2c)

    res = lambda shape: pl.BlockSpec(shape, lambda i: (0, 0))
    return pl.pallas_call(
        _conv_mlp_kernel,
        out_shape=jax.ShapeDtypeStruct((B, OUT), jnp.float32),
        grid=(n_tiles,),
        in_specs=[smem, smem, smem, smem,
                  pl.BlockSpec((tile_b, F), lambda i: (i, 0)),
                  res((256, C1 * 128)),
                  res((NM * 256, 128)),
                  res((768, H1)),
                  res((1, H1)),
                  res((H1, OUT)),
                  res((1, OUT))],
        out_specs=pl.BlockSpec((tile_b, OUT), lambda i: (i, 0)),
        scratch_shapes=[pltpu.VMEM((tile_b, XW), jnp.float32),
                        pltpu.VMEM((tile_b, NM * CS), jnp.float32),
                        pltpu.VMEM((tile_b, K2 * CS), jnp.float32),
                        pltpu.VMEM((tile_b, NB * 128), jnp.float32)],
        compiler_params=pltpu.CompilerParams(
            dimension_semantics=("parallel",)),
    )(w1c, b1, w2c, b2, x, m1, m2, w3p, b3r, w4t, b4r)


# speedup vs baseline: 1.3194x; 1.0176x over previous
import jax
import jax.numpy as jnp
from jax.experimental import pallas as pl
from jax.experimental.pallas import tpu as pltpu

F = 671
H1 = 128
OUT = 30
C1 = 6
NV = 2
NM = C1 - NV
K1 = 5
K2 = 3
NEG = 0.01
NB = 6
XW = 1024
CS = 896
TILE_B = 1024


def _leaky(v):
    return jnp.maximum(v, NEG * v)


def _weights_kernel(w1_ref, w2_ref, m1_ref, m2_ref):
    t = jax.lax.broadcasted_iota(jnp.int32, (256, 128), 0)
    l = jax.lax.broadcasted_iota(jnp.int32, (256, 128), 1)
    d = t - l
    for oc in range(C1):
        m = jnp.where(d == 62, w1_ref[oc, 0], 0.0)
        for k in range(1, K1):
            m = m + jnp.where(d == 62 + k, w1_ref[oc, k], 0.0)
        m1_ref[:, 128 * oc:128 * (oc + 1)] = m
    for i in range(NM):
        m = jnp.where(d == 63, w2_ref[NV + i, 0], 0.0)
        for k in range(1, K2):
            m = m + jnp.where(d == 63 + k, w2_ref[NV + i, k], 0.0)
        m2_ref[256 * i:256 * (i + 1), :] = m


def _conv_mlp_kernel(w1_ref, b1_ref, w2_ref, b2_ref,
                     x_ref, m1_ref, m2_ref,
                     w3_ref, b3_ref, w4_ref, b4_ref,
                     o_ref, xext, a1s, sks, y2s):
    tb = xext.shape[0]

    xext[:, 0:128] = jnp.zeros((tb, 128), jnp.float32)
    xext[:, 128 + F:XW] = jnp.zeros((tb, XW - 128 - F), jnp.float32)
    xext[:, 128:128 + F] = x_ref[...]

    for c in range(NM):
        a1s[:, c * CS + 768:(c + 1) * CS] = jnp.zeros((tb, 128), jnp.float32)
    for k in range(K2):
        sks[:, k * CS + 768:(k + 1) * CS] = jnp.zeros((tb, 128), jnp.float32)

    for j in range(NB):
        r = jnp.dot(xext[:, 128 * j:128 * j + 256], m1_ref[...],
                    preferred_element_type=jnp.float32)
        if j == 0 or j == NB - 1:
            f = jax.lax.broadcasted_iota(jnp.int32, (tb, 128), 1) + (128 * j - 64)
            valid = (f >= 0) & (f < F)
        else:
            valid = None

        def act(c):
            blk = _leaky(r[:, 128 * c:128 * (c + 1)] + b1_ref[c])
            if valid is not None:
                blk = jnp.where(valid, blk, 0.0)
            return blk

        a0 = act(0)
        a1 = act(1)
        for k in range(K2):
            sks[:, k * CS + 128 * j:k * CS + 128 * (j + 1)] = (
                w2_ref[0, k] * a0 + w2_ref[1, k] * a1)
        for c in range(NM):
            a1s[:, c * CS + 128 * j:c * CS + 128 * (j + 1)] = act(NV + c)

    for j in range(NB):
        acc = jnp.dot(a1s[:, 128 * j:128 * j + 256], m2_ref[0:256, :],
                      preferred_element_type=jnp.float32)
        for c in range(1, NM):
            acc = acc + jnp.dot(
                a1s[:, c * CS + 128 * j:c * CS + 128 * j + 256],
                m2_ref[c * 256:(c + 1) * 256, :],
                preferred_element_type=jnp.float32)
        for k in range(K2):
            acc = acc + sks[:, k * CS + 128 * j + 63 + k:
                            k * CS + 128 * j + 63 + k + 128]
        y2s[:, 128 * j:128 * (j + 1)] = _leaky(acc + b2_ref[0])

    y3 = _leaky(jnp.dot(y2s[...], w3_ref[...],
                        preferred_element_type=jnp.float32) + b3_ref[...])
    o_ref[...] = jnp.dot(y3, w4_ref[...],
                         preferred_element_type=jnp.float32) + b4_ref[...]


def kernel(x, w1c, b1, w2c, b2, w3p, b3r, w4t, b4r):
    x = x.astype(jnp.float32)
    B = x.shape[0]
    tile_b = min(TILE_B, B)
    n_tiles = pl.cdiv(B, tile_b)

    smem = pl.BlockSpec(memory_space=pltpu.MemorySpace.SMEM)
    m1, m2 = pl.pallas_call(
        _weights_kernel,
        out_shape=(jax.ShapeDtypeStruct((256, C1 * 128), jnp.float32),
                   jax.ShapeDtypeStruct((NM * 256, 128), jnp.float32)),
        in_specs=[smem, smem],
    )(w1c, w2c)

    res = lambda shape: pl.BlockSpec(shape, lambda i: (0, 0))
    return pl.pallas_call(
        _conv_mlp_kernel,
        out_shape=jax.ShapeDtypeStruct((B, OUT), jnp.float32),
        grid=(n_tiles,),
        in_specs=[smem, smem, smem, smem,
                  pl.BlockSpec((tile_b, F), lambda i: (i, 0)),
                  res((256, C1 * 128)),
                  res((NM * 256, 128)),
                  res((768, H1)),
                  res((1, H1)),
                  res((H1, OUT)),
                  res((1, OUT))],
        out_specs=pl.BlockSpec((tile_b, OUT), lambda i: (i, 0)),
        scratch_shapes=[pltpu.VMEM((tile_b, XW), jnp.float32),
                        pltpu.VMEM((tile_b, NM * CS), jnp.float32),
                        pltpu.VMEM((tile_b, K2 * CS), jnp.float32),
                        pltpu.VMEM((tile_b, NB * 128), jnp.float32)],
        compiler_params=pltpu.CompilerParams(
            dimension_semantics=("parallel",)),
    )(w1c, b1, w2c, b2, x, m1, m2, w3p, b3r, w4t, b4r)
